# baseline (speedup 1.0000x reference)
import jax
import jax.numpy as jnp
from jax.experimental import pallas as pl
from jax.experimental.pallas import tpu as pltpu
from jax.sharding import Mesh, PartitionSpec as P


def _prep_kernel(x_ref, w_ref, b_ref, o_ref):
    o_ref[...] = (
        jnp.dot(x_ref[...], w_ref[...], preferred_element_type=jnp.float32)
        + b_ref[...]
    )


def _layer1_kernel(adj_ref, y1_ref, w2_ref, b2_ref, y2_ref):
    h = jnp.dot(adj_ref[...], y1_ref[...], preferred_element_type=jnp.float32)
    h = jnp.maximum(h, 0.0)
    y2_ref[...] = (
        jnp.dot(h, w2_ref[...], preferred_element_type=jnp.float32) + b2_ref[...]
    )


def _layer2_kernel(adj_ref, y2_ref, w3_ref, b3_ref, w4_ref, b4_ref,
                   emb_ref, z_ref):
    emb = jnp.dot(adj_ref[...], y2_ref[...], preferred_element_type=jnp.float32)
    emb_ref[...] = emb
    t = jnp.maximum(
        jnp.dot(emb, w3_ref[...], preferred_element_type=jnp.float32)
        + b3_ref[...],
        0.0,
    )
    z_ref[...] = (
        jnp.dot(t, w4_ref[...], preferred_element_type=jnp.float32) + b4_ref[...]
    )


def _pick_bm(n, target=400):
    best = None
    for bm in range(8, min(n, target) + 1, 8):
        if n % bm == 0:
            best = bm
    return best if best is not None else n


def _gcn_pipeline(x, adj, W1, b1r, W2, b2r, W3, b3r, W4, b4r):
    n, _ = x.shape
    n_loc = adj.shape[0]
    hid = W1.shape[1]
    emb_d = W2.shape[1]
    proj = W4.shape[1]
    f32 = jnp.float32

    y1 = pl.pallas_call(
        _prep_kernel,
        out_shape=jax.ShapeDtypeStruct((n, hid), f32),
    )(x, W1, b1r)

    bm = _pick_bm(n_loc)
    grid = (n_loc // bm,)

    adj_spec = pl.BlockSpec((bm, n), lambda i: (i, 0))
    full_rhs = lambda d: pl.BlockSpec((n, d), lambda i: (0, 0))
    small = lambda r, c: pl.BlockSpec((r, c), lambda i: (0, 0))
    row_out = lambda d: pl.BlockSpec((bm, d), lambda i: (i, 0))

    y2_loc = pl.pallas_call(
        _layer1_kernel,
        grid=grid,
        in_specs=[
            adj_spec,
            full_rhs(hid),
            small(hid, emb_d),
            small(1, emb_d),
        ],
        out_specs=row_out(emb_d),
        out_shape=jax.ShapeDtypeStruct((n_loc, emb_d), f32),
        compiler_params=pltpu.CompilerParams(
            dimension_semantics=("arbitrary",),
        ),
    )(adj, y1, W2, b2r)

    if n_loc != n:
        y2 = jax.lax.all_gather(y2_loc, "d", axis=0, tiled=True)
    else:
        y2 = y2_loc

    emb_loc, z_loc = pl.pallas_call(
        _layer2_kernel,
        grid=grid,
        in_specs=[
            adj_spec,
            full_rhs(emb_d),
            small(emb_d, proj),
            small(1, proj),
            small(proj, proj),
            small(1, proj),
        ],
        out_specs=[row_out(emb_d), row_out(proj)],
        out_shape=[
            jax.ShapeDtypeStruct((n_loc, emb_d), f32),
            jax.ShapeDtypeStruct((n_loc, proj), f32),
        ],
        compiler_params=pltpu.CompilerParams(
            dimension_semantics=("arbitrary",),
        ),
    )(adj, y2, W3, b3r, W4, b4r)

    return z_loc, emb_loc


@jax.jit
def kernel(x, Adj_, W1, b1, W2, b2, W3, b3, W4, b4):
    n = x.shape[0]

    b1r = b1.reshape(1, -1)
    b2r = b2.reshape(1, -1)
    b3r = b3.reshape(1, -1)
    b4r = b4.reshape(1, -1)

    devs = jax.devices()
    ndev = len(devs)
    while ndev > 1 and n % ndev != 0:
        ndev -= 1

    if ndev == 1:
        z, emb = _gcn_pipeline(x, Adj_, W1, b1r, W2, b2r, W3, b3r, W4, b4r)
        return (z, emb)

    mesh = Mesh(tuple(devs[:ndev]), ("d",))
    rep = P(None, None)
    rows = P("d", None)
    sharded = jax.shard_map(
        _gcn_pipeline,
        mesh=mesh,
        in_specs=(rep, rows, rep, rep, rep, rep, rep, rep, rep, rep),
        out_specs=(rows, rows),
        check_vma=False,
    )
    z, emb = sharded(x, Adj_, W1, b1r, W2, b2r, W3, b3r, W4, b4r)
    return (z, emb)


# speedup vs baseline: 3.6460x; 3.6460x over previous
import jax
import jax.numpy as jnp
from jax.experimental import pallas as pl
from jax.experimental.pallas import tpu as pltpu


def _pick_bm(n, target=400):
    best = None
    for bm in range(8, min(n, target) + 1, 8):
        if n % bm == 0:
            best = bm
    return best if best is not None else n


def _make_fused_kernel(bm):
    def _fused(x_ref, adj_ref, w1_ref, b1_ref, w2_ref, b2_ref,
               w3_ref, b3_ref, w4_ref, b4_ref,
               emb_ref, z_ref, y1_s, y2_s):
        p = pl.program_id(0)
        i = pl.program_id(1)
        f32 = jnp.float32

        @pl.when(jnp.logical_and(p == 0, i == 0))
        def _():
            y1_s[...] = (
                jnp.dot(x_ref[...], w1_ref[...], preferred_element_type=f32)
                + b1_ref[...]
            )

        @pl.when(p == 0)
        def _():
            h = jnp.dot(adj_ref[...], y1_s[...], preferred_element_type=f32)
            h = jnp.maximum(h, 0.0)
            y2_s[pl.ds(i * bm, bm), :] = (
                jnp.dot(h, w2_ref[...], preferred_element_type=f32)
                + b2_ref[...]
            )

        @pl.when(p == 1)
        def _():
            emb = jnp.dot(adj_ref[...], y2_s[...], preferred_element_type=f32)
            emb_ref[...] = emb
            t = jnp.maximum(
                jnp.dot(emb, w3_ref[...], preferred_element_type=f32)
                + b3_ref[...],
                0.0,
            )
            z_ref[...] = (
                jnp.dot(t, w4_ref[...], preferred_element_type=f32)
                + b4_ref[...]
            )

    return _fused


@jax.jit
def kernel(x, Adj_, W1, b1, W2, b2, W3, b3, W4, b4):
    n, in_dim = x.shape
    hid = W1.shape[1]
    emb_d = W2.shape[1]
    proj = W4.shape[1]
    f32 = jnp.float32

    b1r = b1.reshape(1, -1)
    b2r = b2.reshape(1, -1)
    b3r = b3.reshape(1, -1)
    b4r = b4.reshape(1, -1)

    bm = _pick_bm(n)
    grid = (2, n // bm)

    const2 = lambda r, c: pl.BlockSpec((r, c), lambda p, i: (0, 0))
    adj_spec = pl.BlockSpec((bm, n), lambda p, i: (i, 0))
    out_spec = lambda d: pl.BlockSpec((bm, d), lambda p, i: (i * p, 0))

    emb, z = pl.pallas_call(
        _make_fused_kernel(bm),
        grid=grid,
        in_specs=[
            const2(n, in_dim),
            adj_spec,
            const2(in_dim, hid),
            const2(1, hid),
            const2(hid, emb_d),
            const2(1, emb_d),
            const2(emb_d, proj),
            const2(1, proj),
            const2(proj, proj),
            const2(1, proj),
        ],
        out_specs=[out_spec(emb_d), out_spec(proj)],
        out_shape=[
            jax.ShapeDtypeStruct((n, emb_d), f32),
            jax.ShapeDtypeStruct((n, proj), f32),
        ],
        scratch_shapes=[
            pltpu.VMEM((n, hid), f32),
            pltpu.VMEM((n, emb_d), f32),
        ],
        compiler_params=pltpu.CompilerParams(
            dimension_semantics=("arbitrary", "arbitrary"),
        ),
    )(x, Adj_, W1, b1r, W2, b2r, W3, b3r, W4, b4r)

    return (z, emb)
